# Initial kernel scaffold; baseline (speedup 1.0000x reference)
import jax
import jax.numpy as jnp
from jax.experimental import pallas as pl


def kernel(center, pos_context, neg_context, in_table, out_table):
    raise NotImplementedError("write your pallas kernel here")


# speedup vs baseline: 4.3292x; 4.3292x over previous
import functools

import jax
import jax.numpy as jnp
from jax import lax
from jax.experimental import pallas as pl
from jax.experimental.pallas import tpu as pltpu
from jax.experimental.pallas import tpu_sc as plsc

_NC = 2
_NS = 16
_NW = _NC * _NS


@functools.partial(jax.jit, static_argnums=(5, 6, 7, 8))
def _gather_all(center, pos, negf, in_table, out_table, V, D, B, NNEG):
    BPW = B // _NW
    mesh = plsc.VectorSubcoreMesh(
        core_axis_name="c", subcore_axis_name="s",
        num_cores=_NC, num_subcores=_NS)

    @functools.partial(
        pl.kernel,
        out_type=(
            jax.ShapeDtypeStruct((B, D), jnp.float32),
            jax.ShapeDtypeStruct((B, D), jnp.float32),
            jax.ShapeDtypeStruct((B * NNEG, D), jnp.float32),
        ),
        mesh=mesh,
        scratch_types=[
            pltpu.VMEM((BPW,), jnp.int32),
            pltpu.VMEM((BPW, D), jnp.float32),
            pltpu.SemaphoreType.DMA,
        ],
        compiler_params=pltpu.CompilerParams(use_tc_tiling_on_sc=False),
    )
    def gather_k(center_h, pos_h, neg_h, in_t, out_t,
                 cemb_h, pemb_h, nemb_h, idx_v, rows_v, sem):
        wid = lax.axis_index("s") * _NC + lax.axis_index("c")
        base = wid * BPW
        pltpu.sync_copy(center_h.at[pl.ds(base, BPW)], idx_v)
        pltpu.async_copy(in_t.at[idx_v], rows_v, sem).wait()
        pltpu.sync_copy(rows_v, cemb_h.at[pl.ds(base, BPW)])
        pltpu.sync_copy(pos_h.at[pl.ds(base, BPW)], idx_v)
        pltpu.async_copy(out_t.at[idx_v], rows_v, sem).wait()
        pltpu.sync_copy(rows_v, pemb_h.at[pl.ds(base, BPW)])
        nbase = base * NNEG

        def nstep(k, carry):
            off = nbase + k * BPW
            pltpu.sync_copy(neg_h.at[pl.ds(off, BPW)], idx_v)
            pltpu.async_copy(out_t.at[idx_v], rows_v, sem).wait()
            pltpu.sync_copy(rows_v, nemb_h.at[pl.ds(off, BPW)])
            return carry

        lax.fori_loop(0, NNEG, nstep, 0)

    return gather_k(center, pos, negf, in_table, out_table)


@functools.partial(jax.jit, static_argnums=(3, 4, 5))
def _loss(cemb, pemb, nemb, B, D, NNEG):
    BB = 512
    grid = B // BB

    def body(c_ref, p_ref, n_ref, out_ref):
        c = c_ref[...]
        p = p_ref[...]
        n = n_ref[...]
        pos_s = jnp.sum(c * p, axis=-1, keepdims=True)
        neg_s = jnp.sum(n * c[:, None, :], axis=-1)

        def ls(x):
            return jnp.minimum(x, 0.0) - jnp.log(1.0 + jnp.exp(-jnp.abs(x)))

        tot = jnp.sum(ls(pos_s)) + jnp.sum(ls(-neg_s))
        i = pl.program_id(0)

        @pl.when(i == 0)
        def _():
            out_ref[0, 0] = 0.0

        out_ref[0, 0] += tot

        @pl.when(i == grid - 1)
        def _():
            out_ref[0, 0] = -out_ref[0, 0] / B

    out = pl.pallas_call(
        body,
        grid=(grid,),
        in_specs=[
            pl.BlockSpec((BB, D), lambda i: (i, 0)),
            pl.BlockSpec((BB, D), lambda i: (i, 0)),
            pl.BlockSpec((BB, NNEG, D), lambda i: (i, 0, 0)),
        ],
        out_specs=pl.BlockSpec(memory_space=pltpu.SMEM),
        out_shape=jax.ShapeDtypeStruct((1, 1), jnp.float32),
    )(cemb, pemb, nemb)
    return out[0, 0]


def kernel(center, pos_context, neg_context, in_table, out_table):
    B = center.shape[0]
    NNEG = neg_context.shape[1]
    V, D = in_table.shape
    c32 = center.astype(jnp.int32)
    p32 = pos_context.astype(jnp.int32)
    n32 = neg_context.astype(jnp.int32).reshape(B * NNEG)
    cemb, pemb, nemb = _gather_all(c32, p32, n32, in_table, out_table,
                                   V, D, B, NNEG)
    return _loss(cemb, pemb, nemb.reshape(B, NNEG, D), B, D, NNEG)
